# Initial kernel scaffold; baseline (speedup 1.0000x reference)
import jax
import jax.numpy as jnp
from jax.experimental import pallas as pl


def kernel(fnn, p_all, n_all):
    raise NotImplementedError("write your pallas kernel here")


# speedup vs baseline: 1.1392x; 1.1392x over previous
import functools

import jax
import jax.numpy as jnp
import numpy as np
from jax import lax
from jax.experimental import pallas as pl
from jax.experimental.pallas import tpu as pltpu
from jax.experimental.pallas import tpu_sc as plsc

_SAMPLE_N = 65536
_N_FACES = 1000000
_NODD = 15625
_NSHIFT = 6
_W = 1024
_ROWS_PER_STEP = 256
_TINY = float(np.finfo(np.float32).tiny)
_KS1 = 42
_KS2 = 42 ^ 0x1BD11BDA
_ROT_A = (13, 15, 26, 6)
_ROT_B = (17, 29, 16, 24)


def _threefry_fold(hi, lo):
    u32 = jnp.uint32
    x0 = hi
    x1 = lo + u32(_KS1)

    def rounds(x0, x1, rots):
        for r in rots:
            x0 = x0 + x1
            x1 = (x1 << u32(r)) | (x1 >> u32(32 - r))
            x1 = x1 ^ x0
        return x0, x1

    x0, x1 = rounds(x0, x1, _ROT_A)
    x0 = x0 + u32(_KS1)
    x1 = x1 + u32((_KS2 + 1) & 0xFFFFFFFF)
    x0, x1 = rounds(x0, x1, _ROT_B)
    x0 = x0 + u32(_KS2)
    x1 = x1 + u32(2)
    x0, x1 = rounds(x0, x1, _ROT_A)
    x1 = x1 + u32(_KS1 + 3)
    x0, x1 = rounds(x0, x1, _ROT_B)
    x0 = x0 + u32(_KS1)
    x1 = x1 + u32((_KS2 + 4) & 0xFFFFFFFF)
    x0, x1 = rounds(x0, x1, _ROT_A)
    x0 = x0 + u32(_KS2)
    x1 = x1 + u32(5)
    return x0 ^ x1


def _winv_body(fnn_ref, out_ref):
    w = jnp.abs(fnn_ref[...])
    s = jnp.sum(w)
    out_ref[...] = 1.0 / (w / s + jnp.float32(1e-20))


def _make_winv(shape2d):
    return pl.pallas_call(
        _winv_body,
        out_shape=jax.ShapeDtypeStruct(shape2d, jnp.float32),
    )


def _sampler_body(winv_ref, out_ref, *, n_faces, w, rows_per_step, nodd,
                  nshift, chunks):
    u32 = jnp.uint32
    r0 = pl.program_id(0).astype(u32) * u32(rows_per_step)
    lane = lax.broadcasted_iota(u32, (1, w), 1)
    s_iota = lax.broadcasted_iota(u32, (8, 1), 0)
    big = jnp.int32(2147483647)

    for rg in range(rows_per_step // 8):
        i_vec = s_iota + (r0 + u32(rg * 8))
        p_half = i_vec * u32(nodd)
        b_lo = p_half << u32(nshift)
        b_hi = p_half >> u32(32 - nshift)

        def body(t, carry):
            acc_val, acc_idx = carry
            c0 = t.astype(u32) * u32(w)
            col = lane + c0
            lo = b_lo + col
            carrybit = (lo < col).astype(u32)
            hi = b_hi + carrybit
            bits = _threefry_fold(hi, lo)
            fb = (bits >> u32(9)) | u32(0x3F800000)
            f = lax.bitcast_convert_type(fb, jnp.float32) - jnp.float32(1.0)
            u = jnp.maximum(f, jnp.float32(_TINY))
            e = -jnp.log(u)
            wv = winv_ref[t, :].reshape(1, w)
            y = e * wv
            take = y < acc_val
            acc_val = jnp.where(take, y, acc_val)
            acc_idx = jnp.where(take, col.astype(jnp.int32), acc_idx)
            return acc_val, acc_idx

        acc0 = (jnp.full((8, w), jnp.inf, jnp.float32),
                jnp.zeros((8, w), jnp.int32))
        acc_val, acc_idx = lax.fori_loop(0, chunks, body, acc0)
        m = jnp.min(acc_val, axis=1, keepdims=True)
        tie = jnp.where(acc_val == m, acc_idx, big)
        out_ref[rg, :] = jnp.min(tie, axis=1)


def _make_sampler(sample_n, n_faces, w, rows_per_step, nodd, nshift):
    chunks = -(-n_faces // w)
    grid = sample_n // rows_per_step
    body = functools.partial(
        _sampler_body, n_faces=n_faces, w=w, rows_per_step=rows_per_step,
        nodd=nodd, nshift=nshift, chunks=chunks)
    return pl.pallas_call(
        body,
        grid=(grid,),
        in_specs=[pl.BlockSpec((chunks, w), lambda i: (0, 0))],
        out_specs=pl.BlockSpec((rows_per_step // 8, 8), lambda i: (i, 0)),
        out_shape=jax.ShapeDtypeStruct((sample_n // 8, 8), jnp.int32),
        compiler_params=pltpu.CompilerParams(
            dimension_semantics=("arbitrary",)),
    )


def _gather_body(samples_hbm, p_hbm, n_hbm, p_out, n_out,
                 idx_v, prow_v, nrow_v, sem, *, rows16, nc):
    wid = lax.axis_index("s") * nc + lax.axis_index("c")
    base16 = wid * rows16
    pltpu.sync_copy(samples_hbm.at[pl.ds(base16, rows16)], idx_v)
    for j in range(rows16):
        pltpu.async_copy(p_hbm.at[idx_v.at[j]],
                         prow_v.at[pl.ds(j * 128, 128)], sem).wait()
        pltpu.async_copy(n_hbm.at[idx_v.at[j]],
                         nrow_v.at[pl.ds(j * 128, 128)], sem).wait()
    base = wid * (rows16 * 128)
    pltpu.sync_copy(prow_v, p_out.at[pl.ds(base, rows16 * 128)])
    pltpu.sync_copy(nrow_v, n_out.at[pl.ds(base, rows16 * 128)])


def _make_gather(sample_n, n_faces):
    info = plsc.get_sparse_core_info()
    nc, ns = info.num_cores, info.num_subcores
    nw = nc * ns
    per_w = sample_n // nw
    rows16 = per_w // 128
    mesh = plsc.VectorSubcoreMesh(core_axis_name="c", subcore_axis_name="s")
    body = functools.partial(_gather_body, rows16=rows16, nc=nc)
    return pl.kernel(
        body,
        out_type=(jax.ShapeDtypeStruct((sample_n, 3), jnp.float32),
                  jax.ShapeDtypeStruct((sample_n, 3), jnp.float32)),
        mesh=mesh,
        compiler_params=pltpu.CompilerParams(use_tc_tiling_on_sc=False),
        scratch_types=[
            pltpu.VMEM((rows16, 128), jnp.int32),
            pltpu.VMEM((per_w, 3), jnp.float32),
            pltpu.VMEM((per_w, 3), jnp.float32),
            pltpu.SemaphoreType.DMA,
        ],
    )


def kernel(fnn, p_all, n_all):
    chunks = -(-_N_FACES // _W)
    pad = chunks * _W - _N_FACES

    winv = _make_winv((1000, 1000))(fnn.reshape(1000, 1000)).reshape(-1)
    winvc = jnp.concatenate([winv, jnp.full((pad,), jnp.inf, jnp.float32)])
    winvc = winvc.reshape(chunks, _W)

    samples = _make_sampler(_SAMPLE_N, _N_FACES, _W, _ROWS_PER_STEP,
                            _NODD, _NSHIFT)(winvc)
    samples2d = samples.reshape(_SAMPLE_N // 128, 128)

    p, n = _make_gather(_SAMPLE_N, _N_FACES)(samples2d, p_all, n_all)
    return (p, n)
